# Initial kernel scaffold; baseline (speedup 1.0000x reference)
import jax
import jax.numpy as jnp
from jax.experimental import pallas as pl


def kernel(a, W, H):
    raise NotImplementedError("write your pallas kernel here")


# speedup vs baseline: 1.7822x; 1.7822x over previous
import jax
import jax.numpy as jnp
from jax.experimental import pallas as pl

_BN = 512


def _matmul_kernel(a_ref, w_ref, o_ref):
    o_ref[...] = jax.lax.dot_general(
        a_ref[...], w_ref[...], (((1,), (1,)), ((), ())),
        preferred_element_type=jnp.float32)


def kernel(a, W, H):
    B, FIN = a.shape
    FOUT = H.shape[0]
    Wmat = jnp.take(W, H, axis=0).astype(jnp.bfloat16)
    ab = a.astype(jnp.bfloat16)
    z = pl.pallas_call(
        _matmul_kernel,
        grid=(FOUT // _BN,),
        in_specs=[pl.BlockSpec((B, FIN), lambda j: (0, 0)),
                  pl.BlockSpec((_BN, FIN), lambda j: (j, 0))],
        out_specs=pl.BlockSpec((B, _BN), lambda j: (0, j)),
        out_shape=jax.ShapeDtypeStruct((B, FOUT), jnp.float32),
    )(ab, Wmat)
    return z


# speedup vs baseline: 605.9120x; 339.9880x over previous
import dataclasses
import functools

import jax
import jax.numpy as jnp
from jax import lax
from jax.experimental import pallas as pl
from jax.experimental.pallas import tpu as pltpu
from jax.experimental.pallas import tpu_sc as plsc

_BN = 512


def _sc_gather(w_packed, H):
    FOUT, FIN = H.shape
    KW = w_packed.shape[0]
    mesh = plsc.VectorSubcoreMesh(core_axis_name="c", subcore_axis_name="s")
    cp = pltpu.CompilerParams()
    if "needs_layout_passes" in pltpu.CompilerParams.__dataclass_fields__:
        cp = dataclasses.replace(cp, needs_layout_passes=False)

    @functools.partial(
        pl.kernel,
        out_type=jax.ShapeDtypeStruct((FOUT, FIN // 2), jnp.int32),
        mesh=mesh,
        scratch_types=[pltpu.VMEM((KW,), jnp.int32)],
        compiler_params=cp,
    )
    def k(w_hbm, h_hbm, o_hbm, w_vmem):
        pltpu.sync_copy(w_hbm, w_vmem)

        def body(h_vmem, o_vmem):
            @pl.loop(0, FIN // 32)
            def _(g):
                c = g * 32
                idx_a = h_vmem[0, pl.ds(c, 16)]
                idx_b = h_vmem[0, pl.ds(c + 16, 16)]
                g_a = plsc.load_gather(w_vmem, [lax.shift_right_logical(idx_a, 1)])
                g_b = plsc.load_gather(w_vmem, [lax.shift_right_logical(idx_b, 1)])
                sh_a = lax.shift_left(jnp.bitwise_and(idx_a, 1), 4)
                sh_b = lax.shift_left(jnp.bitwise_and(idx_b, 1), 4)
                bits_a = jnp.bitwise_and(lax.shift_right_logical(g_a, sh_a), 0xFFFF)
                bits_b = lax.shift_left(lax.shift_right_logical(g_b, sh_b), 16)
                o_vmem[0, pl.ds(g * 16, 16)] = jnp.bitwise_or(bits_a, bits_b)

        pltpu.emit_pipeline(
            body,
            grid=(FOUT,),
            in_specs=[pl.BlockSpec((1, FIN), lambda i: (i, 0))],
            out_specs=[pl.BlockSpec((1, FIN // 2), lambda i: (i, 0))],
            core_axis_name=("c", "s"),
            dimension_semantics=(pltpu.PARALLEL,),
        )(h_hbm, o_hbm)

    return k(w_packed, H)


def _matmul_kernel(a_ref, w_ref, o_ref):
    o_ref[...] = lax.dot_general(
        a_ref[...], w_ref[...], (((1,), (1,)), ((), ())),
        preferred_element_type=jnp.float32)


def kernel(a, W, H):
    B, FIN = a.shape
    FOUT = H.shape[0]
    wb = W.astype(jnp.bfloat16)
    w_packed = lax.bitcast_convert_type(wb.reshape(-1, 2), jnp.int32)
    wmat_i32 = _sc_gather(w_packed, H)
    wmat = lax.bitcast_convert_type(wmat_i32, jnp.bfloat16).reshape(FOUT, FIN)
    a_perm = (a.reshape(B, FIN // 32, 2, 16)
                .transpose(0, 1, 3, 2)
                .reshape(B, FIN)
                .astype(jnp.bfloat16))
    z = pl.pallas_call(
        _matmul_kernel,
        grid=(FOUT // _BN,),
        in_specs=[pl.BlockSpec((B, FIN), lambda j: (0, 0)),
                  pl.BlockSpec((_BN, FIN), lambda j: (j, 0))],
        out_specs=pl.BlockSpec((B, _BN), lambda j: (0, j)),
        out_shape=jax.ShapeDtypeStruct((B, FOUT), jnp.float32),
    )(a_perm, wmat)
    return z


# speedup vs baseline: 3092.9436x; 5.1046x over previous
import dataclasses
import functools

import jax
import jax.numpy as jnp
from jax import lax
from jax.experimental import pallas as pl
from jax.experimental.pallas import tpu as pltpu
from jax.experimental.pallas import tpu_sc as plsc

_BN = 512


def _sc_gather(w_packed, H):
    FOUT, FIN = H.shape
    KW = w_packed.shape[0]
    mesh = plsc.VectorSubcoreMesh(core_axis_name="c", subcore_axis_name="s")
    cp = pltpu.CompilerParams()
    if "needs_layout_passes" in pltpu.CompilerParams.__dataclass_fields__:
        cp = dataclasses.replace(cp, needs_layout_passes=False)

    @functools.partial(
        pl.kernel,
        out_type=jax.ShapeDtypeStruct((FOUT, FIN // 2), jnp.int32),
        mesh=mesh,
        scratch_types=[pltpu.VMEM((KW,), jnp.int32)],
        compiler_params=cp,
    )
    def k(w_hbm, h_hbm, o_hbm, w_vmem):
        pltpu.sync_copy(w_hbm, w_vmem)

        def body(h_vmem, o_vmem):
            @plsc.parallel_loop(0, FIN // 32, unroll=4)
            def _(g):
                c = g * 32
                idx_a = h_vmem[0, pl.ds(c, 16)]
                idx_b = h_vmem[0, pl.ds(c + 16, 16)]
                g_a = plsc.load_gather(w_vmem, [lax.shift_right_logical(idx_a, 1)])
                g_b = plsc.load_gather(w_vmem, [lax.shift_right_logical(idx_b, 1)])
                sh_a = lax.shift_left(jnp.bitwise_and(idx_a, 1), 4)
                sh_b = lax.shift_left(jnp.bitwise_and(idx_b, 1), 4)
                bits_a = jnp.bitwise_and(lax.shift_right_logical(g_a, sh_a), 0xFFFF)
                bits_b = lax.shift_left(lax.shift_right_logical(g_b, sh_b), 16)
                o_vmem[0, pl.ds(g * 16, 16)] = jnp.bitwise_or(bits_a, bits_b)

        pltpu.emit_pipeline(
            body,
            grid=(FOUT,),
            in_specs=[pl.BlockSpec((1, FIN), lambda i: (i, 0))],
            out_specs=[pl.BlockSpec((1, FIN // 2), lambda i: (i, 0))],
            core_axis_name=("c", "s"),
            dimension_semantics=(pltpu.PARALLEL,),
        )(h_hbm, o_hbm)

    return k(w_packed, H)


def _matmul_kernel(al_ref, ah_ref, w_ref, o_ref):
    wi = w_ref[...]
    w_lo = lax.bitcast_convert_type(
        lax.shift_left(wi, 16), jnp.float32).astype(jnp.bfloat16)
    w_hi = lax.bitcast_convert_type(
        jnp.bitwise_and(wi, jnp.int32(-65536)), jnp.float32).astype(jnp.bfloat16)
    dn = (((1,), (1,)), ((), ()))
    o_ref[...] = (
        lax.dot_general(al_ref[...], w_lo, dn, preferred_element_type=jnp.float32)
        + lax.dot_general(ah_ref[...], w_hi, dn, preferred_element_type=jnp.float32))


def kernel(a, W, H):
    B, FIN = a.shape
    FOUT = H.shape[0]
    wb = W.astype(jnp.bfloat16)
    w_packed = lax.bitcast_convert_type(wb.reshape(-1, 2), jnp.int32)
    wmat_i32 = _sc_gather(w_packed, H)
    a_sp = a.reshape(B, FIN // 32, 2, 16).astype(jnp.bfloat16)
    a_lo = a_sp[:, :, 0, :].reshape(B, FIN // 2)
    a_hi = a_sp[:, :, 1, :].reshape(B, FIN // 2)
    z = pl.pallas_call(
        _matmul_kernel,
        grid=(FOUT // _BN,),
        in_specs=[pl.BlockSpec((B, FIN // 2), lambda j: (0, 0)),
                  pl.BlockSpec((B, FIN // 2), lambda j: (0, 0)),
                  pl.BlockSpec((_BN, FIN // 2), lambda j: (j, 0))],
        out_specs=pl.BlockSpec((B, _BN), lambda j: (0, j)),
        out_shape=jax.ShapeDtypeStruct((B, FOUT), jnp.float32),
    )(a_lo, a_hi, wmat_i32)
    return z


# speedup vs baseline: 3484.0852x; 1.1265x over previous
import dataclasses
import functools

import jax
import jax.numpy as jnp
from jax import lax
from jax.experimental import pallas as pl
from jax.experimental.pallas import tpu as pltpu
from jax.experimental.pallas import tpu_sc as plsc

_BN = 512


def _sc_gather(w_packed, H):
    FOUT, FIN = H.shape
    KW = w_packed.shape[0]
    mesh = plsc.VectorSubcoreMesh(core_axis_name="c", subcore_axis_name="s")
    cp = pltpu.CompilerParams()
    if "needs_layout_passes" in pltpu.CompilerParams.__dataclass_fields__:
        cp = dataclasses.replace(cp, needs_layout_passes=False)

    @functools.partial(
        pl.kernel,
        out_type=jax.ShapeDtypeStruct((FOUT, FIN // 2), jnp.int32),
        mesh=mesh,
        scratch_types=[pltpu.VMEM((KW,), jnp.int32)],
        compiler_params=cp,
    )
    def k(w_hbm, h_hbm, o_hbm, w_vmem):
        pltpu.sync_copy(w_hbm, w_vmem)

        def body(h_vmem, o_vmem):
            for r in range(2):
                @plsc.parallel_loop(0, FIN // 32, unroll=8)
                def _(g, r=r):
                    c = g * 32
                    idx_a = h_vmem[r, pl.ds(c, 16)]
                    idx_b = h_vmem[r, pl.ds(c + 16, 16)]
                    g_a = plsc.load_gather(w_vmem, [lax.shift_right_logical(idx_a, 1)])
                    g_b = plsc.load_gather(w_vmem, [lax.shift_right_logical(idx_b, 1)])
                    sh_a = lax.shift_left(jnp.bitwise_and(idx_a, 1), 4)
                    sh_b = lax.shift_left(jnp.bitwise_and(idx_b, 1), 4)
                    bits_a = jnp.bitwise_and(lax.shift_right_logical(g_a, sh_a), 0xFFFF)
                    bits_b = lax.shift_left(lax.shift_right_logical(g_b, sh_b), 16)
                    o_vmem[r, pl.ds(g * 16, 16)] = jnp.bitwise_or(bits_a, bits_b)

        pltpu.emit_pipeline(
            body,
            grid=(FOUT // 2,),
            in_specs=[pl.BlockSpec((2, FIN), lambda i: (i, 0))],
            out_specs=[pl.BlockSpec((2, FIN // 2), lambda i: (i, 0))],
            core_axis_name=("c", "s"),
            dimension_semantics=(pltpu.PARALLEL,),
        )(h_hbm, o_hbm)

    return k(w_packed, H)


def _matmul_kernel(al_ref, ah_ref, w_ref, o_ref):
    wi = w_ref[...]
    w_lo = lax.bitcast_convert_type(
        lax.shift_left(wi, 16), jnp.float32).astype(jnp.bfloat16)
    w_hi = lax.bitcast_convert_type(wi, jnp.float32).astype(jnp.bfloat16)
    dn = (((1,), (1,)), ((), ()))
    o_ref[...] = (
        lax.dot_general(al_ref[...], w_lo, dn, preferred_element_type=jnp.float32)
        + lax.dot_general(ah_ref[...], w_hi, dn, preferred_element_type=jnp.float32))


def kernel(a, W, H):
    B, FIN = a.shape
    FOUT = H.shape[0]
    wb = W.astype(jnp.bfloat16)
    w_packed = lax.bitcast_convert_type(wb.reshape(-1, 2), jnp.int32)
    wmat_i32 = _sc_gather(w_packed, H)
    a_sp = a.reshape(B, FIN // 32, 2, 16).astype(jnp.bfloat16)
    a_lo = a_sp[:, :, 0, :].reshape(B, FIN // 2)
    a_hi = a_sp[:, :, 1, :].reshape(B, FIN // 2)
    z = pl.pallas_call(
        _matmul_kernel,
        grid=(FOUT // _BN,),
        in_specs=[pl.BlockSpec((B, FIN // 2), lambda j: (0, 0)),
                  pl.BlockSpec((B, FIN // 2), lambda j: (0, 0)),
                  pl.BlockSpec((_BN, FIN // 2), lambda j: (j, 0))],
        out_specs=pl.BlockSpec((B, _BN), lambda j: (0, j)),
        out_shape=jax.ShapeDtypeStruct((B, FOUT), jnp.float32),
    )(a_lo, a_hi, wmat_i32)
    return z
